# baseline (speedup 1.0000x reference)
import functools

import jax
import jax.numpy as jnp
from jax import lax
from jax.experimental import pallas as pl
from jax.experimental.pallas import tpu as pltpu
from jax.experimental.pallas import tpu_sc as plsc

_NE = 8
_K = 2
_DM = 1024
_DFF = 4096
_NTOK = 4096
_F = 4
_BW1 = _DFF // 2
_BW2 = _DM // 2

_NW = 32
_OUT_ELEMS = _NTOK * _K * _DM
_PER_W_OUT = _OUT_ELEMS // _NW
_ZBUF = 16384

_INTERPRET = False


def _sc_zero_body(out_hbm, zbuf):
    c = lax.axis_index("c")
    s = lax.axis_index("s")
    wid = s * 2 + c
    zero = jnp.zeros((16,), jnp.float32)
    for i in range(_ZBUF // 16):
        zbuf[pl.ds(i * 16, 16)] = zero
    for t in range(_PER_W_OUT // _ZBUF):
        pltpu.sync_copy(zbuf, out_hbm.at[pl.ds(wid * _PER_W_OUT + t * _ZBUF, _ZBUF)])


def _sc_zero_fill():
    mesh = plsc.VectorSubcoreMesh(core_axis_name="c", subcore_axis_name="s")
    k = functools.partial(
        pl.kernel,
        mesh=mesh,
        out_type=jax.ShapeDtypeStruct((_OUT_ELEMS,), jnp.float32),
        scratch_types=[pltpu.VMEM((_ZBUF,), jnp.float32)],
    )(_sc_zero_body)
    return k().reshape(_NTOK * _K, _DM)


def _gelu_exact(x):
    z = x * 0.7071067811865476
    a = jnp.abs(z)
    t = 1.0 / (1.0 + 0.3275911 * a)
    poly = t * (0.254829592 + t * (-0.284496736 + t * (1.421413741
               + t * (-1.453152027 + t * 1.061405429))))
    erf_abs = 1.0 - poly * jnp.exp(-a * a)
    erf = jnp.where(z < 0, -erf_abs, erf_abs)
    return 0.5 * x * (1.0 + erf)


def _ffn_body(se_ref, hs_ref, w1_ref, w2_ref, top_out, loads_ref,
              counts_s, h_ref, y_ref, top_ref):
    e = pl.program_id(0)
    f = pl.program_id(1)

    @pl.when((e == 0) & (f == 0))
    def _init():
        se = se_ref[...]
        loads = jnp.zeros((1, _NE), jnp.float32)
        lanes = jax.lax.broadcasted_iota(jnp.int32, (1, _NE), 1)
        for ee in range(_NE):
            c = jnp.sum((se == ee).astype(jnp.int32))
            counts_s[ee] = c
            loads = loads + c.astype(jnp.float32) * (lanes == ee).astype(jnp.float32)
        loads_ref[...] = loads / float(_NTOK)
        top_ref[...] = jnp.zeros_like(top_ref)

    def _acc(j, c):
        return c + (counts_s[j] > 0).astype(jnp.int32)
    npres = jax.lax.fori_loop(0, e + 1, _acc, 0)
    r = npres - 1
    p = counts_s[e] > 0
    b_idx = r // _K

    for half in (0, 1):
        @pl.when(f == half)
        def _w1_phase():
            rows8 = hs_ref[...]
            rowmask = (jax.lax.broadcasted_iota(jnp.int32, (8, 1), 0) == b_idx)
            x = jnp.sum(rows8 * rowmask.astype(rows8.dtype), axis=0, keepdims=True)
            w1 = w1_ref[0]
            hh = jax.lax.dot_general(x, w1, (((1,), (1,)), ((), ())),
                                     preferred_element_type=jnp.float32)
            h_ref[:, half * _BW1:(half + 1) * _BW1] = _gelu_exact(hh)

    for half in (0, 1):
        @pl.when(f == 2 + half)
        def _w2_phase():
            w2 = w2_ref[0]
            yy = jax.lax.dot_general(h_ref[...], w2, (((1,), (1,)), ((), ())),
                                     preferred_element_type=jnp.float32)
            y_ref[:, half * _BW2:(half + 1) * _BW2] = yy

    @pl.when(f == _F - 1)
    def _finish_expert():
        wm = (jax.lax.broadcasted_iota(jnp.int32, (_NE, 1), 0) == r) & p
        top_ref[...] = top_ref[...] + y_ref[...] * wm.astype(jnp.float32)

    @pl.when((e == _NE - 1) & (f == _F - 1))
    def _final():
        top_out[...] = top_ref[...]


def _merge_body(big_ref, top_ref, out_ref):
    del big_ref
    out_ref[...] = top_ref[...]


def kernel(hidden_states, selected_experts, expert_masks, W1, W2):
    del expert_masks
    zeroed = _sc_zero_fill()
    se2d = selected_experts.reshape((_NTOK * _K) // 128, 128)
    top8, loads2d = pl.pallas_call(
        _ffn_body,
        grid=(_NE, _F),
        in_specs=[
            pl.BlockSpec(((_NTOK * _K) // 128, 128), lambda e, f: (0, 0)),
            pl.BlockSpec((8, _DM), lambda e, f: (0, 0)),
            pl.BlockSpec((1, _BW1, _DM),
                         lambda e, f: (e, jnp.minimum(f, 1), 0)),
            pl.BlockSpec((1, _BW2, _DFF),
                         lambda e, f: (e, jnp.maximum(f - 2, 0), 0)),
        ],
        out_specs=[
            pl.BlockSpec((_NE, _DM), lambda e, f: (0, 0)),
            pl.BlockSpec((1, _NE), lambda e, f: (0, 0)),
        ],
        out_shape=[
            jax.ShapeDtypeStruct((_NE, _DM), jnp.float32),
            jax.ShapeDtypeStruct((1, _NE), jnp.float32),
        ],
        scratch_shapes=[
            pltpu.SMEM((_NE,), jnp.int32),
            pltpu.VMEM((1, _DFF), jnp.float32),
            pltpu.VMEM((1, _DM), jnp.float32),
            pltpu.VMEM((_NE, _DM), jnp.float32),
        ],
        interpret=_INTERPRET,
    )(se2d, hidden_states, W1, W2)
    out2d = pl.pallas_call(
        _merge_body,
        grid=(1,),
        in_specs=[
            pl.BlockSpec(memory_space=pl.ANY),
            pl.BlockSpec((_NE, _DM), lambda i: (0, 0)),
        ],
        out_specs=pl.BlockSpec((_NE, _DM), lambda i: (0, 0)),
        out_shape=jax.ShapeDtypeStruct((_NTOK * _K, _DM), jnp.float32),
        input_output_aliases={0: 0},
        interpret=_INTERPRET,
    )(zeroed, top8)
    return out2d.reshape(_NTOK, _K, _DM), loads2d.reshape(_NE)


# speedup vs baseline: 1.3105x; 1.3105x over previous
import jax
import jax.numpy as jnp
from jax.experimental import pallas as pl
from jax.experimental.pallas import tpu as pltpu

_NE = 8
_K = 2
_DM = 1024
_DFF = 4096
_NTOK = 4096
_F = 4
_BW1 = _DFF // 2
_BW2 = _DM // 2
_NB = _NE * _F
_BR = (_NTOK * _K) // _NB

_INTERPRET = False


def _gelu_exact(x):
    z = x * 0.7071067811865476
    a = jnp.abs(z)
    t = 1.0 / (1.0 + 0.3275911 * a)
    poly = t * (0.254829592 + t * (-0.284496736 + t * (1.421413741
               + t * (-1.453152027 + t * 1.061405429))))
    erf_abs = 1.0 - poly * jnp.exp(-a * a)
    erf = jnp.where(z < 0, -erf_abs, erf_abs)
    return 0.5 * x * (1.0 + erf)


def _ffn_body(se_ref, hs_ref, w1_ref, w2_ref, out_ref, loads_ref,
              counts_s, h_ref, y_ref, top_ref):
    e = pl.program_id(0)
    f = pl.program_id(1)

    @pl.when((e == 0) & (f == 0))
    def _init():
        se = se_ref[...]
        loads = jnp.zeros((1, _NE), jnp.float32)
        lanes = jax.lax.broadcasted_iota(jnp.int32, (1, _NE), 1)
        for ee in range(_NE):
            c = jnp.sum((se == ee).astype(jnp.int32))
            counts_s[ee] = c
            loads = loads + c.astype(jnp.float32) * (lanes == ee).astype(jnp.float32)
        loads_ref[...] = loads / float(_NTOK)
        top_ref[...] = jnp.zeros_like(top_ref)

    def _acc(j, c):
        return c + (counts_s[j] > 0).astype(jnp.int32)
    npres = jax.lax.fori_loop(0, e + 1, _acc, 0)
    r = npres - 1
    p = counts_s[e] > 0
    b_idx = r // _K

    for half in (0, 1):
        @pl.when(f == half)
        def _w1_phase():
            rows8 = hs_ref[...]
            rowmask = (jax.lax.broadcasted_iota(jnp.int32, (8, 1), 0) == b_idx)
            x = jnp.sum(rows8 * rowmask.astype(rows8.dtype), axis=0, keepdims=True)
            w1 = w1_ref[0].astype(jnp.bfloat16)
            hh = jax.lax.dot_general(x.astype(jnp.bfloat16), w1,
                                     (((1,), (1,)), ((), ())),
                                     preferred_element_type=jnp.float32)
            h_ref[:, half * _BW1:(half + 1) * _BW1] = _gelu_exact(hh)

    for half in (0, 1):
        @pl.when(f == 2 + half)
        def _w2_phase():
            w2 = w2_ref[0].astype(jnp.bfloat16)
            yy = jax.lax.dot_general(h_ref[...].astype(jnp.bfloat16), w2,
                                     (((1,), (1,)), ((), ())),
                                     preferred_element_type=jnp.float32)
            y_ref[:, half * _BW2:(half + 1) * _BW2] = yy

    @pl.when(f == _F - 1)
    def _finish_expert():
        wm = (jax.lax.broadcasted_iota(jnp.int32, (_NE, 1), 0) == r) & p
        top_ref[...] = top_ref[...] + y_ref[...] * wm.astype(jnp.float32)

    out_ref[...] = jnp.zeros_like(out_ref)

    @pl.when((e == _NE - 1) & (f == _F - 1))
    def _final():
        out_ref[0:_NE, :] = top_ref[...]


def kernel(hidden_states, selected_experts, expert_masks, W1, W2):
    del expert_masks
    se2d = selected_experts.reshape((_NTOK * _K) // 128, 128)
    out2d, loads2d = pl.pallas_call(
        _ffn_body,
        grid=(_NE, _F),
        in_specs=[
            pl.BlockSpec(((_NTOK * _K) // 128, 128), lambda e, f: (0, 0)),
            pl.BlockSpec((8, _DM), lambda e, f: (0, 0)),
            pl.BlockSpec((1, _BW1, _DM),
                         lambda e, f: (e, jnp.minimum(f, 1), 0)),
            pl.BlockSpec((1, _BW2, _DFF),
                         lambda e, f: (e, jnp.maximum(f - 2, 0), 0)),
        ],
        out_specs=[
            pl.BlockSpec((_BR, _DM), lambda e, f: (_NB - 1 - (e * _F + f), 0)),
            pl.BlockSpec((1, _NE), lambda e, f: (0, 0)),
        ],
        out_shape=[
            jax.ShapeDtypeStruct((_NTOK * _K, _DM), jnp.float32),
            jax.ShapeDtypeStruct((1, _NE), jnp.float32),
        ],
        scratch_shapes=[
            pltpu.SMEM((_NE,), jnp.int32),
            pltpu.VMEM((1, _DFF), jnp.float32),
            pltpu.VMEM((1, _DM), jnp.float32),
            pltpu.VMEM((_NE, _DM), jnp.float32),
        ],
        interpret=_INTERPRET,
    )(se2d, hidden_states, W1, W2)
    return out2d.reshape(_NTOK, _K, _DM), loads2d.reshape(_NE)
